# Initial kernel scaffold; baseline (speedup 1.0000x reference)
import jax
import jax.numpy as jnp
from jax.experimental import pallas as pl


def kernel(x):
    raise NotImplementedError("write your pallas kernel here")


# speedup vs baseline: 1.2920x; 1.2920x over previous
import functools

import jax
import jax.numpy as jnp
from jax import lax
from jax.experimental import pallas as pl
from jax.experimental.pallas import tpu as pltpu
from jax.experimental.pallas import tpu_sc as plsc

_B, _C, _N, _M = 8, 3, 16384, 1024
_NC, _NS, _L = 2, 16, 16
_BIG = 1e9


def _make_fps(B, C, N, M, interpret=False):
    SPB = (_NC * _NS) // B
    NLOC = N // SPB
    CHUNKS = NLOC // _L
    BG = B // _NC
    L = _L

    def body(x_hbm, out_hbm, xv, dist, outb, stage, cand, ptmp, shared):
        cid = lax.axis_index("c")
        sid = lax.axis_index("s")
        bg = sid // SPB
        w = sid % SPB
        b = cid * BG + bg
        base = w * NLOC

        lane_i = lax.iota(jnp.int32, L)
        lane_f = lane_i.astype(jnp.float32)
        zeros_i = jnp.zeros((L,), jnp.int32)
        lane0 = lane_i == 0
        row4 = jnp.minimum(lane_i, SPB - 1)
        valid4 = lane_i < SPB

        for c in range(C):
            pltpu.sync_copy(x_hbm.at[pl.ds((b * C + c) * N + base, NLOC)],
                            xv.at[pl.ds(c * NLOC, NLOC)])
            pltpu.sync_copy(x_hbm.at[pl.ds((b * C + c) * N, L)],
                            ptmp.at[pl.ds((c + 1) * L, L)])

        p0 = plsc.load_gather(ptmp, [zeros_i + L])
        p1 = plsc.load_gather(ptmp, [zeros_i + 2 * L])
        p2 = plsc.load_gather(ptmp, [zeros_i + 3 * L])

        plsc.store_scatter(outb, [zeros_i], p0, mask=lane0)
        plsc.store_scatter(outb, [zeros_i + M], p1, mask=lane0)
        plsc.store_scatter(outb, [zeros_i + 2 * M], p2, mask=lane0)

        inf16 = jnp.full((L,), jnp.inf, jnp.float32)

        def init_chunk(i, carry):
            dist[pl.ds(i * L, L)] = inf16
            return carry

        lax.fori_loop(0, CHUNKS, init_chunk, 0)

        basef = base.astype(jnp.float32)
        neg16 = jnp.full((L,), -jnp.inf, jnp.float32)
        zero16 = jnp.zeros((L,), jnp.float32)

        def step(s, carry):
            q0, q1, q2 = carry

            def chunk(i, acc):
                bv, bi = acc
                off = i * L
                t0 = xv[pl.ds(off, L)] - q0
                t1 = xv[pl.ds(NLOC + off, L)] - q1
                t2 = xv[pl.ds(2 * NLOC + off, L)] - q2
                d = t0 * t0 + t1 * t1
                d = d + t2 * t2
                nd = jnp.minimum(dist[pl.ds(off, L)], d)
                dist[pl.ds(off, L)] = nd
                iv = lane_f + (base + off).astype(jnp.float32)
                m = nd > bv
                bv = jnp.where(m, nd, bv)
                bi = jnp.where(m, iv, bi)
                return (bv, bi)

            bv, bi = lax.fori_loop(0, CHUNKS, chunk, (neg16, zero16))

            lm = jnp.max(bv)
            li = jnp.min(jnp.where(bv == lm, bi, _BIG))
            loff = jnp.full((L,), (li - basef).astype(jnp.int32), jnp.int32)
            lpx = plsc.load_gather(xv, [loff])
            lpy = plsc.load_gather(xv, [loff + NLOC])
            lpz = plsc.load_gather(xv, [loff + 2 * NLOC])
            rec = jnp.where(lane_i == 0, lm,
                  jnp.where(lane_i == 1, li,
                  jnp.where(lane_i == 2, lpx,
                  jnp.where(lane_i == 3, lpy,
                  jnp.where(lane_i == 4, lpz, 0.0)))))
            stage[...] = rec

            par = s % 2
            slot = ((par * BG + bg) * SPB + w) * L
            pltpu.sync_copy(stage, shared.at[pl.ds(slot, L)])
            plsc.subcore_barrier()
            gslot = (par * BG + bg) * SPB * L
            pltpu.sync_copy(shared.at[pl.ds(gslot, SPB * L)], cand)

            rbase = row4 * L
            vals = plsc.load_gather(cand, [rbase])
            idxs = plsc.load_gather(cand, [rbase + 1])
            pxs = plsc.load_gather(cand, [rbase + 2])
            pys = plsc.load_gather(cand, [rbase + 3])
            pzs = plsc.load_gather(cand, [rbase + 4])
            gm = jnp.max(jnp.where(valid4, vals, -jnp.inf))
            gi = jnp.min(jnp.where(valid4 & (vals == gm), idxs, _BIG))
            wmask = valid4 & (idxs == gi)
            px = jnp.max(jnp.where(wmask, pxs, -jnp.inf))
            py = jnp.max(jnp.where(wmask, pys, -jnp.inf))
            pz = jnp.max(jnp.where(wmask, pzs, -jnp.inf))

            nq0 = jnp.full((L,), px, jnp.float32)
            nq1 = jnp.full((L,), py, jnp.float32)
            nq2 = jnp.full((L,), pz, jnp.float32)
            ks = jnp.full((L,), s + 1, jnp.int32)
            plsc.store_scatter(outb, [ks], nq0, mask=lane0)
            plsc.store_scatter(outb, [ks + M], nq1, mask=lane0)
            plsc.store_scatter(outb, [ks + 2 * M], nq2, mask=lane0)
            return (nq0, nq1, nq2)

        lax.fori_loop(0, M - 1, step, (p0, p1, p2))

        @pl.when(w == 0)
        def _():
            for c in range(C):
                pltpu.sync_copy(outb.at[pl.ds(c * M, M)],
                                out_hbm.at[pl.ds((b * C + c) * M, M)])

    return pl.kernel(
        body,
        out_type=jax.ShapeDtypeStruct((B * C * M,), jnp.float32),
        mesh=plsc.VectorSubcoreMesh(
            core_axis_name="c", subcore_axis_name="s",
            num_cores=_NC, num_subcores=_NS),
        scratch_types=[
            pltpu.VMEM((C * NLOC,), jnp.float32),
            pltpu.VMEM((NLOC,), jnp.float32),
            pltpu.VMEM((C * M,), jnp.float32),
            pltpu.VMEM((L,), jnp.float32),
            pltpu.VMEM((SPB * L,), jnp.float32),
            pltpu.VMEM(((C + 1) * L,), jnp.float32),
            pltpu.VMEM_SHARED((2 * BG * SPB * L,), jnp.float32),
        ],
        compiler_params=pltpu.CompilerParams(needs_layout_passes=False),
        interpret=interpret,
    )


_fps = _make_fps(_B, _C, _N, _M)


def kernel(x):
    return _fps(x.reshape(-1)).reshape(_B, _C, _M)


# speedup vs baseline: 1.3430x; 1.0395x over previous
import functools

import jax
import jax.numpy as jnp
from jax import lax
from jax.experimental import pallas as pl
from jax.experimental.pallas import tpu as pltpu
from jax.experimental.pallas import tpu_sc as plsc

_B, _C, _N, _M = 8, 3, 16384, 1024
_NC, _NS, _L = 2, 16, 16
_BIG = 1e9


def _make_fps(B, C, N, M, interpret=False):
    SPB = (_NC * _NS) // B
    NLOC = N // SPB
    CHUNKS = NLOC // _L
    BG = B // _NC
    L = _L

    def body(x_hbm, out_hbm, xv, dist, outb, stage, cand, ptmp, shared):
        cid = lax.axis_index("c")
        sid = lax.axis_index("s")
        bg = sid // SPB
        w = sid % SPB
        b = cid * BG + bg
        base = w * NLOC

        lane_i = lax.iota(jnp.int32, L)
        lane_f = lane_i.astype(jnp.float32)
        zeros_i = jnp.zeros((L,), jnp.int32)
        lane0 = lane_i == 0
        row4 = jnp.minimum(lane_i, SPB - 1)
        valid4 = lane_i < SPB

        for c in range(C):
            pltpu.sync_copy(x_hbm.at[pl.ds((b * C + c) * N + base, NLOC)],
                            xv.at[pl.ds(c * NLOC, NLOC)])
            pltpu.sync_copy(x_hbm.at[pl.ds((b * C + c) * N, L)],
                            ptmp.at[pl.ds((c + 1) * L, L)])

        p0 = plsc.load_gather(ptmp, [zeros_i + L])
        p1 = plsc.load_gather(ptmp, [zeros_i + 2 * L])
        p2 = plsc.load_gather(ptmp, [zeros_i + 3 * L])

        plsc.store_scatter(outb, [zeros_i], p0, mask=lane0)
        plsc.store_scatter(outb, [zeros_i + M], p1, mask=lane0)
        plsc.store_scatter(outb, [zeros_i + 2 * M], p2, mask=lane0)

        inf16 = jnp.full((L,), jnp.inf, jnp.float32)

        def init_chunk(i, carry):
            dist[pl.ds(i * L, L)] = inf16
            return carry

        lax.fori_loop(0, CHUNKS, init_chunk, 0)

        basef = base.astype(jnp.float32)
        neg16 = jnp.full((L,), -jnp.inf, jnp.float32)
        zero16 = jnp.zeros((L,), jnp.float32)

        UNROLL = 4

        def step(s, carry):
            q0, q1, q2 = carry

            def chunk(i, acc):
                acc = list(acc)
                for u in range(UNROLL):
                    bv, bi = acc[2 * u], acc[2 * u + 1]
                    off = (i * UNROLL + u) * L
                    t0 = xv[pl.ds(off, L)] - q0
                    t1 = xv[pl.ds(NLOC + off, L)] - q1
                    t2 = xv[pl.ds(2 * NLOC + off, L)] - q2
                    d = t0 * t0 + t1 * t1
                    d = d + t2 * t2
                    nd = jnp.minimum(dist[pl.ds(off, L)], d)
                    dist[pl.ds(off, L)] = nd
                    iv = lane_f + (base + off).astype(jnp.float32)
                    m = nd > bv
                    acc[2 * u] = jnp.where(m, nd, bv)
                    acc[2 * u + 1] = jnp.where(m, iv, bi)
                return tuple(acc)

            acc = lax.fori_loop(0, CHUNKS // UNROLL, chunk,
                                (neg16, zero16) * UNROLL)

            bv, bi = acc[0], acc[1]
            for u in range(1, UNROLL):
                ov, oi = acc[2 * u], acc[2 * u + 1]
                m = (ov > bv) | ((ov == bv) & (oi < bi))
                bv = jnp.where(m, ov, bv)
                bi = jnp.where(m, oi, bi)

            lm = jnp.max(bv)
            li = jnp.min(jnp.where(bv == lm, bi, _BIG))
            loff = jnp.full((L,), (li - basef).astype(jnp.int32), jnp.int32)
            lpx = plsc.load_gather(xv, [loff])
            lpy = plsc.load_gather(xv, [loff + NLOC])
            lpz = plsc.load_gather(xv, [loff + 2 * NLOC])
            rec = jnp.where(lane_i == 0, lm,
                  jnp.where(lane_i == 1, li,
                  jnp.where(lane_i == 2, lpx,
                  jnp.where(lane_i == 3, lpy,
                  jnp.where(lane_i == 4, lpz, 0.0)))))
            stage[...] = rec

            par = s % 2
            slot = ((par * BG + bg) * SPB + w) * L
            pltpu.sync_copy(stage, shared.at[pl.ds(slot, L)])
            plsc.subcore_barrier()
            gslot = (par * BG + bg) * SPB * L
            pltpu.sync_copy(shared.at[pl.ds(gslot, SPB * L)], cand)

            rbase = row4 * L
            vals = plsc.load_gather(cand, [rbase])
            idxs = plsc.load_gather(cand, [rbase + 1])
            pxs = plsc.load_gather(cand, [rbase + 2])
            pys = plsc.load_gather(cand, [rbase + 3])
            pzs = plsc.load_gather(cand, [rbase + 4])
            gm = jnp.max(jnp.where(valid4, vals, -jnp.inf))
            gi = jnp.min(jnp.where(valid4 & (vals == gm), idxs, _BIG))
            wmask = valid4 & (idxs == gi)
            px = jnp.max(jnp.where(wmask, pxs, -jnp.inf))
            py = jnp.max(jnp.where(wmask, pys, -jnp.inf))
            pz = jnp.max(jnp.where(wmask, pzs, -jnp.inf))

            nq0 = jnp.full((L,), px, jnp.float32)
            nq1 = jnp.full((L,), py, jnp.float32)
            nq2 = jnp.full((L,), pz, jnp.float32)
            ks = jnp.full((L,), s + 1, jnp.int32)
            plsc.store_scatter(outb, [ks], nq0, mask=lane0)
            plsc.store_scatter(outb, [ks + M], nq1, mask=lane0)
            plsc.store_scatter(outb, [ks + 2 * M], nq2, mask=lane0)
            return (nq0, nq1, nq2)

        lax.fori_loop(0, M - 1, step, (p0, p1, p2))

        @pl.when(w == 0)
        def _():
            for c in range(C):
                pltpu.sync_copy(outb.at[pl.ds(c * M, M)],
                                out_hbm.at[pl.ds((b * C + c) * M, M)])

    return pl.kernel(
        body,
        out_type=jax.ShapeDtypeStruct((B * C * M,), jnp.float32),
        mesh=plsc.VectorSubcoreMesh(
            core_axis_name="c", subcore_axis_name="s",
            num_cores=_NC, num_subcores=_NS),
        scratch_types=[
            pltpu.VMEM((C * NLOC,), jnp.float32),
            pltpu.VMEM((NLOC,), jnp.float32),
            pltpu.VMEM((C * M,), jnp.float32),
            pltpu.VMEM((L,), jnp.float32),
            pltpu.VMEM((SPB * L,), jnp.float32),
            pltpu.VMEM(((C + 1) * L,), jnp.float32),
            pltpu.VMEM_SHARED((2 * BG * SPB * L,), jnp.float32),
        ],
        compiler_params=pltpu.CompilerParams(needs_layout_passes=False),
        interpret=interpret,
    )


_fps = _make_fps(_B, _C, _N, _M)


def kernel(x):
    return _fps(x.reshape(-1)).reshape(_B, _C, _M)


# speedup vs baseline: 3.7524x; 2.7940x over previous
import functools

import jax
import jax.numpy as jnp
from jax import lax
from jax.experimental import pallas as pl
from jax.experimental.pallas import tpu as pltpu
from jax.experimental.pallas import tpu_sc as plsc

_B, _C, _N, _M = 8, 3, 16384, 1024
_NC, _NS, _L = 2, 16, 16
_BIG = 1e9


def _make_fps(B, C, N, M, interpret=False):
    SPB = (_NC * _NS) // B
    NLOC = N // SPB
    CHUNKS = NLOC // _L
    BG = B // _NC
    L = _L

    def body(x_hbm, out_hbm, xv, dist, outb, stage, cand, ptmp, shared):
        cid = lax.axis_index("c")
        sid = lax.axis_index("s")
        bg = sid // SPB
        w = sid % SPB
        b = cid * BG + bg
        base = w * NLOC

        lane_i = lax.iota(jnp.int32, L)
        lane_f = lane_i.astype(jnp.float32)
        zeros_i = jnp.zeros((L,), jnp.int32)
        lane0 = lane_i == 0
        row4 = jnp.minimum(lane_i, SPB - 1)
        valid4 = lane_i < SPB

        for c in range(C):
            pltpu.sync_copy(x_hbm.at[pl.ds((b * C + c) * N + base, NLOC)],
                            xv.at[pl.ds(c * NLOC, NLOC)])
            pltpu.sync_copy(x_hbm.at[pl.ds((b * C + c) * N, L)],
                            ptmp.at[pl.ds((c + 1) * L, L)])

        p0 = plsc.load_gather(ptmp, [zeros_i + L])
        p1 = plsc.load_gather(ptmp, [zeros_i + 2 * L])
        p2 = plsc.load_gather(ptmp, [zeros_i + 3 * L])

        plsc.store_scatter(outb, [zeros_i], p0, mask=lane0)
        plsc.store_scatter(outb, [zeros_i + M], p1, mask=lane0)
        plsc.store_scatter(outb, [zeros_i + 2 * M], p2, mask=lane0)

        inf16 = jnp.full((L,), jnp.inf, jnp.float32)

        def init_chunk(i, carry):
            dist[pl.ds(i * L, L)] = inf16
            return carry

        lax.fori_loop(0, CHUNKS, init_chunk, 0)

        basef = base.astype(jnp.float32)
        neg16 = jnp.full((L,), -jnp.inf, jnp.float32)
        zero16 = jnp.zeros((L,), jnp.float32)

        UNROLL = 4

        def step(s, carry):
            q0, q1, q2 = carry

            def chunk(i, acc):
                acc = list(acc)
                for u in range(UNROLL):
                    bv, bi = acc[2 * u], acc[2 * u + 1]
                    off = (i * UNROLL + u) * L
                    t0 = xv[pl.ds(off, L)] - q0
                    t1 = xv[pl.ds(NLOC + off, L)] - q1
                    t2 = xv[pl.ds(2 * NLOC + off, L)] - q2
                    d = t0 * t0 + t1 * t1
                    d = d + t2 * t2
                    nd = jnp.minimum(dist[pl.ds(off, L)], d)
                    dist[pl.ds(off, L)] = nd
                    iv = lane_f + (base + off).astype(jnp.float32)
                    m = nd > bv
                    acc[2 * u] = jnp.where(m, nd, bv)
                    acc[2 * u + 1] = jnp.where(m, iv, bi)
                return tuple(acc)

            acc = plsc.parallel_loop(
                0, CHUNKS // UNROLL, step=1, unroll=2,
                carry=(neg16, zero16) * UNROLL)(chunk)

            bv, bi = acc[0], acc[1]
            for u in range(1, UNROLL):
                ov, oi = acc[2 * u], acc[2 * u + 1]
                m = (ov > bv) | ((ov == bv) & (oi < bi))
                bv = jnp.where(m, ov, bv)
                bi = jnp.where(m, oi, bi)

            lm = jnp.max(bv)
            li = jnp.min(jnp.where(bv == lm, bi, _BIG))
            loff = jnp.full((L,), (li - basef).astype(jnp.int32), jnp.int32)
            lpx = plsc.load_gather(xv, [loff])
            lpy = plsc.load_gather(xv, [loff + NLOC])
            lpz = plsc.load_gather(xv, [loff + 2 * NLOC])
            rec = jnp.where(lane_i == 0, lm,
                  jnp.where(lane_i == 1, li,
                  jnp.where(lane_i == 2, lpx,
                  jnp.where(lane_i == 3, lpy,
                  jnp.where(lane_i == 4, lpz, 0.0)))))
            stage[...] = rec

            par = s % 2
            slot = ((par * BG + bg) * SPB + w) * L
            pltpu.sync_copy(stage, shared.at[pl.ds(slot, L)])
            plsc.subcore_barrier()
            gslot = (par * BG + bg) * SPB * L
            pltpu.sync_copy(shared.at[pl.ds(gslot, SPB * L)], cand)

            rbase = row4 * L
            vals = plsc.load_gather(cand, [rbase])
            idxs = plsc.load_gather(cand, [rbase + 1])
            pxs = plsc.load_gather(cand, [rbase + 2])
            pys = plsc.load_gather(cand, [rbase + 3])
            pzs = plsc.load_gather(cand, [rbase + 4])
            gm = jnp.max(jnp.where(valid4, vals, -jnp.inf))
            gi = jnp.min(jnp.where(valid4 & (vals == gm), idxs, _BIG))
            wmask = valid4 & (idxs == gi)
            px = jnp.max(jnp.where(wmask, pxs, -jnp.inf))
            py = jnp.max(jnp.where(wmask, pys, -jnp.inf))
            pz = jnp.max(jnp.where(wmask, pzs, -jnp.inf))

            nq0 = jnp.full((L,), px, jnp.float32)
            nq1 = jnp.full((L,), py, jnp.float32)
            nq2 = jnp.full((L,), pz, jnp.float32)
            ks = jnp.full((L,), s + 1, jnp.int32)
            plsc.store_scatter(outb, [ks], nq0, mask=lane0)
            plsc.store_scatter(outb, [ks + M], nq1, mask=lane0)
            plsc.store_scatter(outb, [ks + 2 * M], nq2, mask=lane0)
            return (nq0, nq1, nq2)

        lax.fori_loop(0, M - 1, step, (p0, p1, p2))

        @pl.when(w == 0)
        def _():
            for c in range(C):
                pltpu.sync_copy(outb.at[pl.ds(c * M, M)],
                                out_hbm.at[pl.ds((b * C + c) * M, M)])

    return pl.kernel(
        body,
        out_type=jax.ShapeDtypeStruct((B * C * M,), jnp.float32),
        mesh=plsc.VectorSubcoreMesh(
            core_axis_name="c", subcore_axis_name="s",
            num_cores=_NC, num_subcores=_NS),
        scratch_types=[
            pltpu.VMEM((C * NLOC,), jnp.float32),
            pltpu.VMEM((NLOC,), jnp.float32),
            pltpu.VMEM((C * M,), jnp.float32),
            pltpu.VMEM((L,), jnp.float32),
            pltpu.VMEM((SPB * L,), jnp.float32),
            pltpu.VMEM(((C + 1) * L,), jnp.float32),
            pltpu.VMEM_SHARED((2 * BG * SPB * L,), jnp.float32),
        ],
        compiler_params=pltpu.CompilerParams(needs_layout_passes=False),
        interpret=interpret,
    )


_fps = _make_fps(_B, _C, _N, _M)


def kernel(x):
    return _fps(x.reshape(-1)).reshape(_B, _C, _M)
